# baseline (speedup 1.0000x reference)
import jax
import jax.numpy as jnp
from jax.experimental import pallas as pl

N = 4097
BR = 64
GRID = (N + BR - 1) // BR


def _relu_body(clp_ref, cup_ref, alow_ref, aup_ref, ocl_ref, ocu_ref):
    i = pl.program_id(0)
    cl = clp_ref[...]
    cu = cup_ref[...]

    dead = cu <= 0.0
    pos = jnp.logical_and(~dead, cl >= 0.0)
    cross = jnp.logical_and(~dead, cl < 0.0)

    alpha = jnp.where(cu < -cl, jnp.float32(1e-5), jnp.float32(1.0))
    denom = jnp.where(cross, cu - cl, jnp.float32(1.0))
    lam = jnp.where(cross, cu / denom, jnp.float32(0.0))

    zero = jnp.float32(0.0)
    one = jnp.float32(1.0)
    diag_low = jnp.where(pos, one, jnp.where(cross, alpha, zero))
    diag_up = jnp.where(pos, one, jnp.where(cross, lam, zero))
    bias_up = jnp.where(cross, -lam * cl, zero)

    rows = i * BR + jax.lax.broadcasted_iota(jnp.int32, (BR, N), 0)
    cols = jax.lax.broadcasted_iota(jnp.int32, (BR, N), 1)
    on_diag = rows == cols

    alow_ref[...] = jnp.where(on_diag, diag_low, zero)

    @pl.when(i == GRID - 1)
    def _():
        aup_ref[...] = jnp.where(
            on_diag, diag_up, jnp.where(rows == N - 1, bias_up, zero)
        )

    @pl.when(i != GRID - 1)
    def _():
        aup_ref[...] = jnp.where(on_diag, diag_up, zero)

    @pl.when(i == 0)
    def _():
        out_cl = jnp.where(pos, cl, jnp.where(cross, alpha * cl, zero))
        out_cu = jnp.where(dead, zero, cu)
        ocl_ref[...] = out_cl[:, : N - 1]
        ocu_ref[...] = out_cu[:, : N - 1]


def kernel(concrete_lower, concrete_upper, abstract_lower_in, abstract_upper_in):
    n = N - 1
    pad = jnp.ones((1, 1), dtype=jnp.float32)
    clp = jnp.concatenate([concrete_lower.reshape(1, n), pad], axis=1)
    cup = jnp.concatenate([concrete_upper.reshape(1, n), pad], axis=1)

    a_low, a_up, out_cl, out_cu = pl.pallas_call(
        _relu_body,
        grid=(GRID,),
        in_specs=[
            pl.BlockSpec((1, N), lambda i: (0, 0)),
            pl.BlockSpec((1, N), lambda i: (0, 0)),
        ],
        out_specs=[
            pl.BlockSpec((BR, N), lambda i: (i, 0)),
            pl.BlockSpec((BR, N), lambda i: (i, 0)),
            pl.BlockSpec((1, n), lambda i: (0, 0)),
            pl.BlockSpec((1, n), lambda i: (0, 0)),
        ],
        out_shape=[
            jax.ShapeDtypeStruct((N, N), jnp.float32),
            jax.ShapeDtypeStruct((N, N), jnp.float32),
            jax.ShapeDtypeStruct((1, n), jnp.float32),
            jax.ShapeDtypeStruct((1, n), jnp.float32),
        ],
    )(clp, cup)
    return (out_cl.reshape(n), out_cu.reshape(n), a_low, a_up)


# speedup vs baseline: 1.1728x; 1.1728x over previous
import jax
import jax.numpy as jnp
from jax.experimental import pallas as pl

N = 4097
BR = 192
GRID = (N + BR - 1) // BR


def _relu_body(clp_ref, cup_ref, alow_ref, aup_ref, ocl_ref, ocu_ref):
    i = pl.program_id(0)
    cl = clp_ref[...]
    cu = cup_ref[...]

    dead = cu <= 0.0
    pos = jnp.logical_and(~dead, cl >= 0.0)
    cross = jnp.logical_and(~dead, cl < 0.0)

    alpha = jnp.where(cu < -cl, jnp.float32(1e-5), jnp.float32(1.0))
    denom = jnp.where(cross, cu - cl, jnp.float32(1.0))
    lam = jnp.where(cross, cu / denom, jnp.float32(0.0))

    zero = jnp.float32(0.0)
    one = jnp.float32(1.0)
    diag_low = jnp.where(pos, one, jnp.where(cross, alpha, zero))
    diag_up = jnp.where(pos, one, jnp.where(cross, lam, zero))
    bias_up = jnp.where(cross, -lam * cl, zero)

    rows = i * BR + jax.lax.broadcasted_iota(jnp.int32, (BR, N), 0)
    cols = jax.lax.broadcasted_iota(jnp.int32, (BR, N), 1)
    on_diag = rows == cols

    alow_ref[...] = jnp.where(on_diag, diag_low, zero)

    @pl.when(i == GRID - 1)
    def _():
        aup_ref[...] = jnp.where(
            on_diag, diag_up, jnp.where(rows == N - 1, bias_up, zero)
        )

    @pl.when(i != GRID - 1)
    def _():
        aup_ref[...] = jnp.where(on_diag, diag_up, zero)

    @pl.when(i == 0)
    def _():
        out_cl = jnp.where(pos, cl, jnp.where(cross, alpha * cl, zero))
        out_cu = jnp.where(dead, zero, cu)
        ocl_ref[...] = out_cl[:, : N - 1]
        ocu_ref[...] = out_cu[:, : N - 1]


def kernel(concrete_lower, concrete_upper, abstract_lower_in, abstract_upper_in):
    n = N - 1
    pad = jnp.ones((1, 1), dtype=jnp.float32)
    clp = jnp.concatenate([concrete_lower.reshape(1, n), pad], axis=1)
    cup = jnp.concatenate([concrete_upper.reshape(1, n), pad], axis=1)

    a_low, a_up, out_cl, out_cu = pl.pallas_call(
        _relu_body,
        grid=(GRID,),
        in_specs=[
            pl.BlockSpec((1, N), lambda i: (0, 0)),
            pl.BlockSpec((1, N), lambda i: (0, 0)),
        ],
        out_specs=[
            pl.BlockSpec((BR, N), lambda i: (i, 0)),
            pl.BlockSpec((BR, N), lambda i: (i, 0)),
            pl.BlockSpec((1, n), lambda i: (0, 0)),
            pl.BlockSpec((1, n), lambda i: (0, 0)),
        ],
        out_shape=[
            jax.ShapeDtypeStruct((N, N), jnp.float32),
            jax.ShapeDtypeStruct((N, N), jnp.float32),
            jax.ShapeDtypeStruct((1, n), jnp.float32),
            jax.ShapeDtypeStruct((1, n), jnp.float32),
        ],
    )(clp, cup)
    return (out_cl.reshape(n), out_cu.reshape(n), a_low, a_up)


# speedup vs baseline: 1.1831x; 1.0088x over previous
import jax
import jax.numpy as jnp
from jax.experimental import pallas as pl

N = 4097
BR = 144
GRID = (N + BR - 1) // BR


def _relu_body(clp_ref, cup_ref, alow_ref, aup_ref, ocl_ref, ocu_ref):
    i = pl.program_id(0)
    cl = clp_ref[...]
    cu = cup_ref[...]

    dead = cu <= 0.0
    pos = jnp.logical_and(~dead, cl >= 0.0)
    cross = jnp.logical_and(~dead, cl < 0.0)

    alpha = jnp.where(cu < -cl, jnp.float32(1e-5), jnp.float32(1.0))
    denom = jnp.where(cross, cu - cl, jnp.float32(1.0))
    lam = jnp.where(cross, cu / denom, jnp.float32(0.0))

    zero = jnp.float32(0.0)
    one = jnp.float32(1.0)
    diag_low = jnp.where(pos, one, jnp.where(cross, alpha, zero))
    diag_up = jnp.where(pos, one, jnp.where(cross, lam, zero))
    bias_up = jnp.where(cross, -lam * cl, zero)

    rows = i * BR + jax.lax.broadcasted_iota(jnp.int32, (BR, N), 0)
    cols = jax.lax.broadcasted_iota(jnp.int32, (BR, N), 1)
    on_diag = rows == cols

    alow_ref[...] = jnp.where(on_diag, diag_low, zero)

    @pl.when(i == GRID - 1)
    def _():
        aup_ref[...] = jnp.where(
            on_diag, diag_up, jnp.where(rows == N - 1, bias_up, zero)
        )

    @pl.when(i != GRID - 1)
    def _():
        aup_ref[...] = jnp.where(on_diag, diag_up, zero)

    @pl.when(i == 0)
    def _():
        out_cl = jnp.where(pos, cl, jnp.where(cross, alpha * cl, zero))
        out_cu = jnp.where(dead, zero, cu)
        ocl_ref[...] = out_cl[:, : N - 1]
        ocu_ref[...] = out_cu[:, : N - 1]


def kernel(concrete_lower, concrete_upper, abstract_lower_in, abstract_upper_in):
    n = N - 1
    pad = jnp.ones((1, 1), dtype=jnp.float32)
    clp = jnp.concatenate([concrete_lower.reshape(1, n), pad], axis=1)
    cup = jnp.concatenate([concrete_upper.reshape(1, n), pad], axis=1)

    a_low, a_up, out_cl, out_cu = pl.pallas_call(
        _relu_body,
        grid=(GRID,),
        in_specs=[
            pl.BlockSpec((1, N), lambda i: (0, 0)),
            pl.BlockSpec((1, N), lambda i: (0, 0)),
        ],
        out_specs=[
            pl.BlockSpec((BR, N), lambda i: (i, 0)),
            pl.BlockSpec((BR, N), lambda i: (i, 0)),
            pl.BlockSpec((1, n), lambda i: (0, 0)),
            pl.BlockSpec((1, n), lambda i: (0, 0)),
        ],
        out_shape=[
            jax.ShapeDtypeStruct((N, N), jnp.float32),
            jax.ShapeDtypeStruct((N, N), jnp.float32),
            jax.ShapeDtypeStruct((1, n), jnp.float32),
            jax.ShapeDtypeStruct((1, n), jnp.float32),
        ],
    )(clp, cup)
    return (out_cl.reshape(n), out_cu.reshape(n), a_low, a_up)
